# Initial kernel scaffold; baseline (speedup 1.0000x reference)
import jax
import jax.numpy as jnp
from jax.experimental import pallas as pl


def kernel(x, edge_index, batch, W1, b1, W2, b2, W3, b3, W4, b4):
    raise NotImplementedError("write your pallas kernel here")


# speedup vs baseline: 11.3571x; 11.3571x over previous
import functools

import jax
import jax.numpy as jnp
from jax import lax
from jax.experimental import pallas as pl
from jax.experimental.pallas import tpu as pltpu
from jax.experimental.pallas import tpu_sc as plsc

N = 10000
E = 320000
D_IN = 128
HID = 64
NUM_CLASSES = 10
NUM_GRAPHS = 64

NC, NS = 2, 16
NPAD = 10240
RPT = NPAD // NS
CH = 80
DW = 128

R = 1000
G = N // R

_mesh = plsc.VectorSubcoreMesh(
    core_axis_name="c", subcore_axis_name="s", num_cores=NC, num_subcores=NS)

f32 = jnp.float32
i32 = jnp.int32


def _deg_body(dst_hbm, zeros_hbm, out0_hbm, out1_hbm, deg_sh, idx_v, ones_v):
    c = lax.axis_index("c")
    s = lax.axis_index("s")
    pltpu.sync_copy(zeros_hbm, deg_sh.at[pl.ds(s * RPT, RPT)])
    for k in range(CH // 16):
        ones_v[pl.ds(k * 16, 16)] = jnp.full((16,), 1.0, f32)
    plsc.subcore_barrier()

    ept = E // (NC * NS)
    base = (c * NS + s) * ept

    def body(j, carry):
        off = base + j * CH
        pltpu.sync_copy(dst_hbm.at[pl.ds(off, CH)], idx_v.at[0])
        pltpu.sync_copy(ones_v, deg_sh.at[idx_v.at[0]], add=True)
        return carry

    lax.fori_loop(0, ept // CH, body, 0)
    plsc.subcore_barrier()

    @pl.when(c == 0)
    def _():
        pltpu.sync_copy(deg_sh.at[pl.ds(s * RPT, RPT)],
                        out0_hbm.at[pl.ds(s * RPT, RPT)])

    @pl.when(c == 1)
    def _():
        pltpu.sync_copy(deg_sh.at[pl.ds(s * RPT, RPT)],
                        out1_hbm.at[pl.ds(s * RPT, RPT)])


_deg_call = pl.kernel(
    _deg_body,
    out_type=(jax.ShapeDtypeStruct((NPAD,), f32),
              jax.ShapeDtypeStruct((NPAD,), f32)),
    mesh=_mesh,
    scratch_types=[
        pltpu.VMEM_SHARED((NPAD,), f32),
        pltpu.VMEM((1, CH), i32),
        pltpu.VMEM((CH,), f32),
    ],
)


def _agg_body(g_hbm, src_hbm, dst_hbm, zeros_hbm,
              out0_hbm, out1_hbm, s_sh, isrc, idst, rows):
    c = lax.axis_index("c")
    s = lax.axis_index("s")
    pltpu.sync_copy(zeros_hbm, s_sh.at[pl.ds(s * RPT, RPT)])
    plsc.subcore_barrier()

    ept = E // (NC * NS)
    base = (c * NS + s) * ept

    def body(j, carry):
        off = base + j * CH
        pltpu.sync_copy(src_hbm.at[pl.ds(off, CH)], isrc.at[0])
        pltpu.sync_copy(dst_hbm.at[pl.ds(off, CH)], idst.at[0])
        pltpu.sync_copy(g_hbm.at[isrc.at[0]], rows)
        pltpu.sync_copy(rows, s_sh.at[idst.at[0]], add=True)
        return carry

    lax.fori_loop(0, ept // CH, body, 0)
    plsc.subcore_barrier()

    @pl.when(c == 0)
    def _():
        pltpu.sync_copy(s_sh.at[pl.ds(s * RPT, RPT)],
                        out0_hbm.at[pl.ds(s * RPT, RPT)])

    @pl.when(c == 1)
    def _():
        pltpu.sync_copy(s_sh.at[pl.ds(s * RPT, RPT)],
                        out1_hbm.at[pl.ds(s * RPT, RPT)])


_agg_call = pl.kernel(
    _agg_body,
    out_type=(jax.ShapeDtypeStruct((NPAD, DW), f32),
              jax.ShapeDtypeStruct((NPAD, DW), f32)),
    mesh=_mesh,
    scratch_types=[
        pltpu.VMEM_SHARED((NPAD, DW), f32),
        pltpu.VMEM((1, CH), i32),
        pltpu.VMEM((1, CH), i32),
        pltpu.VMEM((CH, DW), f32),
    ],
)


def _k1_body(x_ref, d0_ref, d1_ref, w_ref, g_ref, dinv_ref):
    deg = d0_ref[...] + d1_ref[...] + 1.0
    dinv = lax.rsqrt(deg)
    h = jnp.dot(x_ref[...], w_ref[...], preferred_element_type=f32)
    g_ref[...] = h * dinv
    dinv_ref[...] = dinv


def _k1_call(x, d0, d1, w):
    return pl.pallas_call(
        _k1_body,
        grid=(G,),
        in_specs=[
            pl.BlockSpec((R, D_IN), lambda i: (i, 0)),
            pl.BlockSpec((R, 1), lambda i: (i, 0)),
            pl.BlockSpec((R, 1), lambda i: (i, 0)),
            pl.BlockSpec((D_IN, 2 * HID), lambda i: (0, 0)),
        ],
        out_specs=[
            pl.BlockSpec((R, DW), lambda i: (i, 0)),
            pl.BlockSpec((R, 1), lambda i: (i, 0)),
        ],
        out_shape=[
            jax.ShapeDtypeStruct((N, DW), f32),
            jax.ShapeDtypeStruct((N, 1), f32),
        ],
    )(x, d0, d1, w)


def _mid_body(dout, s0, s1, g, dinv, w, b, o_ref):
    dv = dinv[...]
    a = jnp.maximum(dv * (s0[...] + s1[...] + g[...]) + b[...], 0.0)
    h = jnp.dot(a, w[...], preferred_element_type=f32)
    g_next = h * dv
    if dout < DW:
        g_next = jnp.concatenate(
            [g_next, jnp.zeros((R, DW - dout), f32)], axis=1)
    o_ref[...] = g_next


def _mid_call(s0, s1, g, dinv, w, b, dout):
    return pl.pallas_call(
        functools.partial(_mid_body, dout),
        grid=(G,),
        in_specs=[
            pl.BlockSpec((R, DW), lambda i: (i, 0)),
            pl.BlockSpec((R, DW), lambda i: (i, 0)),
            pl.BlockSpec((R, DW), lambda i: (i, 0)),
            pl.BlockSpec((R, 1), lambda i: (i, 0)),
            pl.BlockSpec((DW, dout), lambda i: (0, 0)),
            pl.BlockSpec((1, DW), lambda i: (0, 0)),
        ],
        out_specs=pl.BlockSpec((R, DW), lambda i: (i, 0)),
        out_shape=jax.ShapeDtypeStruct((N, DW), f32),
    )(s0, s1, g, dinv, w, b.reshape(1, -1))


def _fin_body(s0, s1, g, dinv, b3, batch, w4, b4, out_ref, sums, cnts):
    i = pl.program_id(0)

    @pl.when(i == 0)
    def _():
        sums[...] = jnp.zeros((NUM_GRAPHS, HID), f32)
        cnts[...] = jnp.zeros((NUM_GRAPHS, 1), f32)

    dv = dinv[...]
    a = dv * (s0[...] + s1[...] + g[...])[:, :HID] + b3[...]
    oh = (batch[...] == lax.broadcasted_iota(i32, (R, NUM_GRAPHS), 1))
    oh = oh.astype(f32)
    sums[...] += lax.dot_general(oh, a, (((0,), (0,)), ((), ())),
                                 preferred_element_type=f32)
    cnts[...] += lax.dot_general(oh, jnp.ones((R, 1), f32),
                                 (((0,), (0,)), ((), ())),
                                 preferred_element_type=f32)

    @pl.when(i == G - 1)
    def _():
        pooled = sums[...] / jnp.maximum(cnts[...], 1.0)
        out_ref[...] = jnp.dot(pooled, w4[...],
                               preferred_element_type=f32) + b4[...]


def _fin_call(s0, s1, g, dinv, b3, batch, w4, b4):
    return pl.pallas_call(
        _fin_body,
        grid=(G,),
        in_specs=[
            pl.BlockSpec((R, DW), lambda i: (i, 0)),
            pl.BlockSpec((R, DW), lambda i: (i, 0)),
            pl.BlockSpec((R, DW), lambda i: (i, 0)),
            pl.BlockSpec((R, 1), lambda i: (i, 0)),
            pl.BlockSpec((1, HID), lambda i: (0, 0)),
            pl.BlockSpec((R, 1), lambda i: (i, 0)),
            pl.BlockSpec((HID, NUM_CLASSES), lambda i: (0, 0)),
            pl.BlockSpec((1, NUM_CLASSES), lambda i: (0, 0)),
        ],
        out_specs=pl.BlockSpec((NUM_GRAPHS, NUM_CLASSES), lambda i: (0, 0)),
        out_shape=jax.ShapeDtypeStruct((NUM_GRAPHS, NUM_CLASSES), f32),
        scratch_shapes=[
            pltpu.VMEM((NUM_GRAPHS, HID), f32),
            pltpu.VMEM((NUM_GRAPHS, 1), f32),
        ],
    )(s0, s1, g, dinv, b3.reshape(1, -1), batch, w4, b4.reshape(1, -1))


def kernel(x, edge_index, batch, W1, b1, W2, b2, W3, b3, W4, b4):
    src = edge_index[0].astype(i32)
    dst = edge_index[1].astype(i32)
    batch2 = batch.astype(i32).reshape(N, 1)
    z1 = jnp.zeros((RPT,), f32)
    zw = jnp.zeros((RPT, DW), f32)

    deg0, deg1 = _deg_call(dst, z1)
    g1, dinv = _k1_call(x, deg0[:N].reshape(N, 1),
                        deg1[:N].reshape(N, 1), W1)

    s1a, s1b = _agg_call(g1, src, dst, zw)
    g2 = _mid_call(s1a[:N], s1b[:N], g1, dinv, W2, b1, 2 * HID)

    s2a, s2b = _agg_call(g2, src, dst, zw)
    g3 = _mid_call(s2a[:N], s2b[:N], g2, dinv, W3, b2, HID)

    s3a, s3b = _agg_call(g3, src, dst, zw)
    out = _fin_call(s3a, s3b, g3, dinv, b3, batch2, W4, b4)
    return out


# speedup vs baseline: 26.2057x; 2.3074x over previous
import functools

import jax
import jax.numpy as jnp
from jax import lax
from jax.experimental import pallas as pl
from jax.experimental.pallas import tpu as pltpu
from jax.experimental.pallas import tpu_sc as plsc

N = 10000
E = 320000
D_IN = 128
HID = 64
NUM_CLASSES = 10
NUM_GRAPHS = 64

NC, NS = 2, 16
NPAD = 10240
RPT = NPAD // NS
CH = 80
NCHT = E // (NC * NS * CH)
DW = 128

R = 1000
G = N // R

_mesh = plsc.VectorSubcoreMesh(
    core_axis_name="c", subcore_axis_name="s", num_cores=NC, num_subcores=NS)

f32 = jnp.float32
i32 = jnp.int32


def _deg_body(dstr_hbm, zeros_hbm, out0_hbm, out1_hbm, deg_sh, idst, ones_v,
              sem):
    c = lax.axis_index("c")
    s = lax.axis_index("s")
    t = c * NS + s
    pltpu.sync_copy(zeros_hbm, deg_sh.at[pl.ds(s * RPT, RPT)])
    pltpu.sync_copy(dstr_hbm.at[t], idst)
    for k in range(CH // 16):
        ones_v[pl.ds(k * 16, 16)] = jnp.full((16,), 1.0, f32)
    plsc.subcore_barrier()

    def body(jj, carry):
        for b in range(8):
            j = jj * 8 + b

            @pl.when(j < NCHT)
            def _():
                pltpu.async_copy(ones_v, deg_sh.at[idst.at[j]], sem, add=True)

        for b in range(8):
            j = jj * 8 + b

            @pl.when(j < NCHT)
            def _():
                pltpu.make_async_copy(
                    ones_v, deg_sh.at[idst.at[j]], sem).wait()

        return carry

    lax.fori_loop(0, (NCHT + 7) // 8, body, 0)
    plsc.subcore_barrier()

    @pl.when(c == 0)
    def _():
        pltpu.sync_copy(deg_sh.at[pl.ds(s * RPT, RPT)],
                        out0_hbm.at[pl.ds(s * RPT, RPT)])

    @pl.when(c == 1)
    def _():
        pltpu.sync_copy(deg_sh.at[pl.ds(s * RPT, RPT)],
                        out1_hbm.at[pl.ds(s * RPT, RPT)])


_deg_call = pl.kernel(
    _deg_body,
    out_type=(jax.ShapeDtypeStruct((NPAD,), f32),
              jax.ShapeDtypeStruct((NPAD,), f32)),
    mesh=_mesh,
    scratch_types=[
        pltpu.VMEM_SHARED((NPAD,), f32),
        pltpu.VMEM((NCHT, CH), i32),
        pltpu.VMEM((CH,), f32),
        pltpu.SemaphoreType.DMA,
    ],
)


def _agg_body(g_hbm, srcf_hbm, dstr_hbm, zeros_hbm,
              out0_hbm, out1_hbm, s_sh, isrc, idst, rows, sem0, sem1):
    c = lax.axis_index("c")
    s = lax.axis_index("s")
    t = c * NS + s
    ept = NCHT * CH
    pltpu.sync_copy(zeros_hbm, s_sh.at[pl.ds(s * RPT, RPT)])
    pltpu.sync_copy(srcf_hbm.at[pl.ds(t * ept, ept)], isrc)
    pltpu.sync_copy(dstr_hbm.at[t], idst)
    plsc.subcore_barrier()

    sems = (sem0, sem1)
    pltpu.async_copy(g_hbm.at[isrc.at[pl.ds(0, CH)]], rows.at[0], sem0)

    def body(jj, carry):
        for b in range(2):
            j = jj * 2 + b
            nxt = j + 1

            @pl.when(nxt < NCHT)
            def _():
                pltpu.async_copy(g_hbm.at[isrc.at[pl.ds(nxt * CH, CH)]],
                                 rows.at[1 - b], sems[1 - b])

            @pl.when(j < NCHT)
            def _():
                pltpu.make_async_copy(
                    g_hbm.at[isrc.at[pl.ds(j * CH, CH)]],
                    rows.at[b], sems[b]).wait()
                pltpu.sync_copy(rows.at[b], s_sh.at[idst.at[j]], add=True)

        return carry

    lax.fori_loop(0, (NCHT + 1) // 2, body, 0)
    plsc.subcore_barrier()

    @pl.when(c == 0)
    def _():
        pltpu.sync_copy(s_sh.at[pl.ds(s * RPT, RPT)],
                        out0_hbm.at[pl.ds(s * RPT, RPT)])

    @pl.when(c == 1)
    def _():
        pltpu.sync_copy(s_sh.at[pl.ds(s * RPT, RPT)],
                        out1_hbm.at[pl.ds(s * RPT, RPT)])


_agg_call = pl.kernel(
    _agg_body,
    out_type=(jax.ShapeDtypeStruct((NPAD, DW), f32),
              jax.ShapeDtypeStruct((NPAD, DW), f32)),
    mesh=_mesh,
    scratch_types=[
        pltpu.VMEM_SHARED((NPAD, DW), f32),
        pltpu.VMEM((NCHT * CH,), i32),
        pltpu.VMEM((NCHT, CH), i32),
        pltpu.VMEM((2, CH, DW), f32),
        pltpu.SemaphoreType.DMA,
        pltpu.SemaphoreType.DMA,
    ],
)


def _k1_body(x_ref, d0_ref, d1_ref, w_ref, g_ref, dinv_ref):
    deg = d0_ref[...] + d1_ref[...] + 1.0
    dinv = lax.rsqrt(deg)
    h = jnp.dot(x_ref[...], w_ref[...], preferred_element_type=f32)
    g_ref[...] = h * dinv
    dinv_ref[...] = dinv


def _k1_call(x, d0, d1, w):
    return pl.pallas_call(
        _k1_body,
        grid=(G,),
        in_specs=[
            pl.BlockSpec((R, D_IN), lambda i: (i, 0)),
            pl.BlockSpec((R, 1), lambda i: (i, 0)),
            pl.BlockSpec((R, 1), lambda i: (i, 0)),
            pl.BlockSpec((D_IN, 2 * HID), lambda i: (0, 0)),
        ],
        out_specs=[
            pl.BlockSpec((R, DW), lambda i: (i, 0)),
            pl.BlockSpec((R, 1), lambda i: (i, 0)),
        ],
        out_shape=[
            jax.ShapeDtypeStruct((N, DW), f32),
            jax.ShapeDtypeStruct((N, 1), f32),
        ],
    )(x, d0, d1, w)


def _mid_body(dout, s0, s1, g, dinv, w, b, o_ref):
    dv = dinv[...]
    a = jnp.maximum(dv * (s0[...] + s1[...] + g[...]) + b[...], 0.0)
    h = jnp.dot(a, w[...], preferred_element_type=f32)
    g_next = h * dv
    if dout < DW:
        g_next = jnp.concatenate(
            [g_next, jnp.zeros((R, DW - dout), f32)], axis=1)
    o_ref[...] = g_next


def _mid_call(s0, s1, g, dinv, w, b, dout):
    return pl.pallas_call(
        functools.partial(_mid_body, dout),
        grid=(G,),
        in_specs=[
            pl.BlockSpec((R, DW), lambda i: (i, 0)),
            pl.BlockSpec((R, DW), lambda i: (i, 0)),
            pl.BlockSpec((R, DW), lambda i: (i, 0)),
            pl.BlockSpec((R, 1), lambda i: (i, 0)),
            pl.BlockSpec((DW, dout), lambda i: (0, 0)),
            pl.BlockSpec((1, DW), lambda i: (0, 0)),
        ],
        out_specs=pl.BlockSpec((R, DW), lambda i: (i, 0)),
        out_shape=jax.ShapeDtypeStruct((N, DW), f32),
    )(s0, s1, g, dinv, w, b.reshape(1, -1))


def _fin_body(s0, s1, g, dinv, b3, batch, w4, b4, out_ref, sums, cnts):
    i = pl.program_id(0)

    @pl.when(i == 0)
    def _():
        sums[...] = jnp.zeros((NUM_GRAPHS, HID), f32)
        cnts[...] = jnp.zeros((NUM_GRAPHS, 1), f32)

    dv = dinv[...]
    a = dv * (s0[...] + s1[...] + g[...])[:, :HID] + b3[...]
    oh = (batch[...] == lax.broadcasted_iota(i32, (R, NUM_GRAPHS), 1))
    oh = oh.astype(f32)
    sums[...] += lax.dot_general(oh, a, (((0,), (0,)), ((), ())),
                                 preferred_element_type=f32)
    cnts[...] += lax.dot_general(oh, jnp.ones((R, 1), f32),
                                 (((0,), (0,)), ((), ())),
                                 preferred_element_type=f32)

    @pl.when(i == G - 1)
    def _():
        pooled = sums[...] / jnp.maximum(cnts[...], 1.0)
        out_ref[...] = jnp.dot(pooled, w4[...],
                               preferred_element_type=f32) + b4[...]


def _fin_call(s0, s1, g, dinv, b3, batch, w4, b4):
    return pl.pallas_call(
        _fin_body,
        grid=(G,),
        in_specs=[
            pl.BlockSpec((R, DW), lambda i: (i, 0)),
            pl.BlockSpec((R, DW), lambda i: (i, 0)),
            pl.BlockSpec((R, DW), lambda i: (i, 0)),
            pl.BlockSpec((R, 1), lambda i: (i, 0)),
            pl.BlockSpec((1, HID), lambda i: (0, 0)),
            pl.BlockSpec((R, 1), lambda i: (i, 0)),
            pl.BlockSpec((HID, NUM_CLASSES), lambda i: (0, 0)),
            pl.BlockSpec((1, NUM_CLASSES), lambda i: (0, 0)),
        ],
        out_specs=pl.BlockSpec((NUM_GRAPHS, NUM_CLASSES), lambda i: (0, 0)),
        out_shape=jax.ShapeDtypeStruct((NUM_GRAPHS, NUM_CLASSES), f32),
        scratch_shapes=[
            pltpu.VMEM((NUM_GRAPHS, HID), f32),
            pltpu.VMEM((NUM_GRAPHS, 1), f32),
        ],
    )(s0, s1, g, dinv, b3.reshape(1, -1), batch, w4, b4.reshape(1, -1))


def kernel(x, edge_index, batch, W1, b1, W2, b2, W3, b3, W4, b4):
    src = edge_index[0].astype(i32)
    dst = edge_index[1].astype(i32).reshape(NC * NS, NCHT, CH)
    batch2 = batch.astype(i32).reshape(N, 1)
    z1 = jnp.zeros((RPT,), f32)
    zw = jnp.zeros((RPT, DW), f32)

    deg0, deg1 = _deg_call(dst, z1)
    g1, dinv = _k1_call(x, deg0[:N].reshape(N, 1),
                        deg1[:N].reshape(N, 1), W1)

    s1a, s1b = _agg_call(g1, src, dst, zw)
    g2 = _mid_call(s1a[:N], s1b[:N], g1, dinv, W2, b1, 2 * HID)

    s2a, s2b = _agg_call(g2, src, dst, zw)
    g3 = _mid_call(s2a[:N], s2b[:N], g2, dinv, W3, b2, HID)

    s3a, s3b = _agg_call(g3, src, dst, zw)
    out = _fin_call(s3a, s3b, g3, dinv, b3, batch2, W4, b4)
    return out


# speedup vs baseline: 26.2255x; 1.0008x over previous
import functools

import jax
import jax.numpy as jnp
from jax import lax
from jax.experimental import pallas as pl
from jax.experimental.pallas import tpu as pltpu
from jax.experimental.pallas import tpu_sc as plsc

N = 10000
E = 320000
D_IN = 128
HID = 64
NUM_CLASSES = 10
NUM_GRAPHS = 64

NC, NS = 2, 16
NPAD = 10240
RPT = NPAD // NS
CH = 80
NCHT = E // (NC * NS * CH)
DW = 128

R = 1000
G = N // R

_mesh = plsc.VectorSubcoreMesh(
    core_axis_name="c", subcore_axis_name="s", num_cores=NC, num_subcores=NS)

f32 = jnp.float32
i32 = jnp.int32


def _deg_body(dstr_hbm, zeros_hbm, out0_hbm, out1_hbm, deg_sh, idst, ones_v,
              sem):
    c = lax.axis_index("c")
    s = lax.axis_index("s")
    t = c * NS + s
    pltpu.sync_copy(zeros_hbm, deg_sh.at[pl.ds(s * RPT, RPT)])
    pltpu.sync_copy(dstr_hbm.at[t], idst)
    for k in range(CH // 16):
        ones_v[pl.ds(k * 16, 16)] = jnp.full((16,), 1.0, f32)
    plsc.subcore_barrier()

    def body(jj, carry):
        for b in range(8):
            j = jj * 8 + b

            @pl.when(j < NCHT)
            def _():
                pltpu.async_copy(ones_v, deg_sh.at[idst.at[j]], sem, add=True)

        for b in range(8):
            j = jj * 8 + b

            @pl.when(j < NCHT)
            def _():
                pltpu.make_async_copy(
                    ones_v, deg_sh.at[idst.at[j]], sem).wait()

        return carry

    lax.fori_loop(0, (NCHT + 7) // 8, body, 0)
    plsc.subcore_barrier()

    @pl.when(c == 0)
    def _():
        pltpu.sync_copy(deg_sh.at[pl.ds(s * RPT, RPT)],
                        out0_hbm.at[pl.ds(s * RPT, RPT)])

    @pl.when(c == 1)
    def _():
        pltpu.sync_copy(deg_sh.at[pl.ds(s * RPT, RPT)],
                        out1_hbm.at[pl.ds(s * RPT, RPT)])


_deg_call = pl.kernel(
    _deg_body,
    out_type=(jax.ShapeDtypeStruct((NPAD,), f32),
              jax.ShapeDtypeStruct((NPAD,), f32)),
    mesh=_mesh,
    scratch_types=[
        pltpu.VMEM_SHARED((NPAD,), f32),
        pltpu.VMEM((NCHT, CH), i32),
        pltpu.VMEM((CH,), f32),
        pltpu.SemaphoreType.DMA,
    ],
)


def _agg_body(g_hbm, srcf_hbm, dstr_hbm, zeros_hbm,
              out0_hbm, out1_hbm, s_sh, isrc, idst, rows,
              sem0, sem1, sem2, sem3):
    c = lax.axis_index("c")
    s = lax.axis_index("s")
    t = c * NS + s
    ept = NCHT * CH
    pltpu.sync_copy(zeros_hbm, s_sh.at[pl.ds(s * RPT, RPT)])
    pltpu.sync_copy(srcf_hbm.at[pl.ds(t * ept, ept)], isrc)
    pltpu.sync_copy(dstr_hbm.at[t], idst)
    plsc.subcore_barrier()

    sems_g = (sem0, sem1)
    sems_s = (sem2, sem3)
    pltpu.async_copy(g_hbm.at[isrc.at[pl.ds(0, CH)]], rows.at[0], sem0)

    def body(jj, carry):
        for b in range(2):
            j = jj * 2 + b
            nxt = j + 1

            @pl.when(nxt < NCHT)
            def _():
                @pl.when(nxt >= 2)
                def _():
                    pltpu.make_async_copy(
                        rows.at[1 - b], s_sh.at[idst.at[nxt - 2]],
                        sems_s[1 - b]).wait()

                pltpu.async_copy(g_hbm.at[isrc.at[pl.ds(nxt * CH, CH)]],
                                 rows.at[1 - b], sems_g[1 - b])

            @pl.when(j < NCHT)
            def _():
                pltpu.make_async_copy(
                    g_hbm.at[isrc.at[pl.ds(j * CH, CH)]],
                    rows.at[b], sems_g[b]).wait()
                pltpu.async_copy(rows.at[b], s_sh.at[idst.at[j]],
                                 sems_s[b], add=True)

        return carry

    lax.fori_loop(0, (NCHT + 1) // 2, body, 0)
    pltpu.make_async_copy(rows.at[(NCHT - 2) % 2],
                          s_sh.at[idst.at[NCHT - 2]],
                          sems_s[(NCHT - 2) % 2]).wait()
    pltpu.make_async_copy(rows.at[(NCHT - 1) % 2],
                          s_sh.at[idst.at[NCHT - 1]],
                          sems_s[(NCHT - 1) % 2]).wait()
    plsc.subcore_barrier()

    @pl.when(c == 0)
    def _():
        pltpu.sync_copy(s_sh.at[pl.ds(s * RPT, RPT)],
                        out0_hbm.at[pl.ds(s * RPT, RPT)])

    @pl.when(c == 1)
    def _():
        pltpu.sync_copy(s_sh.at[pl.ds(s * RPT, RPT)],
                        out1_hbm.at[pl.ds(s * RPT, RPT)])


_agg_call = pl.kernel(
    _agg_body,
    out_type=(jax.ShapeDtypeStruct((NPAD, DW), f32),
              jax.ShapeDtypeStruct((NPAD, DW), f32)),
    mesh=_mesh,
    scratch_types=[
        pltpu.VMEM_SHARED((NPAD, DW), f32),
        pltpu.VMEM((NCHT * CH,), i32),
        pltpu.VMEM((NCHT, CH), i32),
        pltpu.VMEM((2, CH, DW), f32),
        pltpu.SemaphoreType.DMA,
        pltpu.SemaphoreType.DMA,
        pltpu.SemaphoreType.DMA,
        pltpu.SemaphoreType.DMA,
    ],
)


def _k1_body(x_ref, d0_ref, d1_ref, w_ref, g_ref, dinv_ref):
    deg = d0_ref[...] + d1_ref[...] + 1.0
    dinv = lax.rsqrt(deg)
    h = jnp.dot(x_ref[...], w_ref[...], preferred_element_type=f32)
    g_ref[...] = h * dinv
    dinv_ref[...] = dinv


def _k1_call(x, d0, d1, w):
    return pl.pallas_call(
        _k1_body,
        grid=(G,),
        in_specs=[
            pl.BlockSpec((R, D_IN), lambda i: (i, 0)),
            pl.BlockSpec((R, 1), lambda i: (i, 0)),
            pl.BlockSpec((R, 1), lambda i: (i, 0)),
            pl.BlockSpec((D_IN, 2 * HID), lambda i: (0, 0)),
        ],
        out_specs=[
            pl.BlockSpec((R, DW), lambda i: (i, 0)),
            pl.BlockSpec((R, 1), lambda i: (i, 0)),
        ],
        out_shape=[
            jax.ShapeDtypeStruct((N, DW), f32),
            jax.ShapeDtypeStruct((N, 1), f32),
        ],
    )(x, d0, d1, w)


def _mid_body(dout, s0, s1, g, dinv, w, b, o_ref):
    dv = dinv[...]
    a = jnp.maximum(dv * (s0[...] + s1[...] + g[...]) + b[...], 0.0)
    h = jnp.dot(a, w[...], preferred_element_type=f32)
    g_next = h * dv
    if dout < DW:
        g_next = jnp.concatenate(
            [g_next, jnp.zeros((R, DW - dout), f32)], axis=1)
    o_ref[...] = g_next


def _mid_call(s0, s1, g, dinv, w, b, dout):
    return pl.pallas_call(
        functools.partial(_mid_body, dout),
        grid=(G,),
        in_specs=[
            pl.BlockSpec((R, DW), lambda i: (i, 0)),
            pl.BlockSpec((R, DW), lambda i: (i, 0)),
            pl.BlockSpec((R, DW), lambda i: (i, 0)),
            pl.BlockSpec((R, 1), lambda i: (i, 0)),
            pl.BlockSpec((DW, dout), lambda i: (0, 0)),
            pl.BlockSpec((1, DW), lambda i: (0, 0)),
        ],
        out_specs=pl.BlockSpec((R, DW), lambda i: (i, 0)),
        out_shape=jax.ShapeDtypeStruct((N, DW), f32),
    )(s0, s1, g, dinv, w, b.reshape(1, -1))


def _fin_body(s0, s1, g, dinv, b3, batch, w4, b4, out_ref, sums, cnts):
    i = pl.program_id(0)

    @pl.when(i == 0)
    def _():
        sums[...] = jnp.zeros((NUM_GRAPHS, HID), f32)
        cnts[...] = jnp.zeros((NUM_GRAPHS, 1), f32)

    dv = dinv[...]
    a = dv * (s0[...] + s1[...] + g[...])[:, :HID] + b3[...]
    oh = (batch[...] == lax.broadcasted_iota(i32, (R, NUM_GRAPHS), 1))
    oh = oh.astype(f32)
    sums[...] += lax.dot_general(oh, a, (((0,), (0,)), ((), ())),
                                 preferred_element_type=f32)
    cnts[...] += lax.dot_general(oh, jnp.ones((R, 1), f32),
                                 (((0,), (0,)), ((), ())),
                                 preferred_element_type=f32)

    @pl.when(i == G - 1)
    def _():
        pooled = sums[...] / jnp.maximum(cnts[...], 1.0)
        out_ref[...] = jnp.dot(pooled, w4[...],
                               preferred_element_type=f32) + b4[...]


def _fin_call(s0, s1, g, dinv, b3, batch, w4, b4):
    return pl.pallas_call(
        _fin_body,
        grid=(G,),
        in_specs=[
            pl.BlockSpec((R, DW), lambda i: (i, 0)),
            pl.BlockSpec((R, DW), lambda i: (i, 0)),
            pl.BlockSpec((R, DW), lambda i: (i, 0)),
            pl.BlockSpec((R, 1), lambda i: (i, 0)),
            pl.BlockSpec((1, HID), lambda i: (0, 0)),
            pl.BlockSpec((R, 1), lambda i: (i, 0)),
            pl.BlockSpec((HID, NUM_CLASSES), lambda i: (0, 0)),
            pl.BlockSpec((1, NUM_CLASSES), lambda i: (0, 0)),
        ],
        out_specs=pl.BlockSpec((NUM_GRAPHS, NUM_CLASSES), lambda i: (0, 0)),
        out_shape=jax.ShapeDtypeStruct((NUM_GRAPHS, NUM_CLASSES), f32),
        scratch_shapes=[
            pltpu.VMEM((NUM_GRAPHS, HID), f32),
            pltpu.VMEM((NUM_GRAPHS, 1), f32),
        ],
    )(s0, s1, g, dinv, b3.reshape(1, -1), batch, w4, b4.reshape(1, -1))


def kernel(x, edge_index, batch, W1, b1, W2, b2, W3, b3, W4, b4):
    src = edge_index[0].astype(i32)
    dst = edge_index[1].astype(i32).reshape(NC * NS, NCHT, CH)
    batch2 = batch.astype(i32).reshape(N, 1)
    z1 = jnp.zeros((RPT,), f32)
    zw = jnp.zeros((RPT, DW), f32)

    deg0, deg1 = _deg_call(dst, z1)
    g1, dinv = _k1_call(x, deg0[:N].reshape(N, 1),
                        deg1[:N].reshape(N, 1), W1)

    s1a, s1b = _agg_call(g1, src, dst, zw)
    g2 = _mid_call(s1a[:N], s1b[:N], g1, dinv, W2, b1, 2 * HID)

    s2a, s2b = _agg_call(g2, src, dst, zw)
    g3 = _mid_call(s2a[:N], s2b[:N], g2, dinv, W3, b2, HID)

    s3a, s3b = _agg_call(g3, src, dst, zw)
    out = _fin_call(s3a, s3b, g3, dinv, b3, batch2, W4, b4)
    return out


# speedup vs baseline: 27.3209x; 1.0418x over previous
import functools

import jax
import jax.numpy as jnp
from jax import lax
from jax.experimental import pallas as pl
from jax.experimental.pallas import tpu as pltpu
from jax.experimental.pallas import tpu_sc as plsc

N = 10000
E = 320000
D_IN = 128
HID = 64
NUM_CLASSES = 10
NUM_GRAPHS = 64

NC, NS = 2, 16
NPAD = 10240
RPT = NPAD // NS
CH = 80
NCHT = E // (NC * NS * CH)
DW = 128

R = 1000
G = N // R

_mesh = plsc.VectorSubcoreMesh(
    core_axis_name="c", subcore_axis_name="s", num_cores=NC, num_subcores=NS)

f32 = jnp.float32
i32 = jnp.int32


def _deg_body(dstr_hbm, zeros_hbm, out0_hbm, out1_hbm, deg_sh, idst, ones_v,
              sem):
    c = lax.axis_index("c")
    s = lax.axis_index("s")
    t = c * NS + s
    pltpu.sync_copy(zeros_hbm, deg_sh.at[pl.ds(s * RPT, RPT)])
    pltpu.sync_copy(dstr_hbm.at[t], idst)
    for k in range(CH // 16):
        ones_v[pl.ds(k * 16, 16)] = jnp.full((16,), 1.0, f32)
    plsc.subcore_barrier()

    def body(jj, carry):
        for b in range(8):
            j = jj * 8 + b

            @pl.when(j < NCHT)
            def _():
                pltpu.async_copy(ones_v, deg_sh.at[idst.at[j]], sem, add=True)

        for b in range(8):
            j = jj * 8 + b

            @pl.when(j < NCHT)
            def _():
                pltpu.make_async_copy(
                    ones_v, deg_sh.at[idst.at[j]], sem).wait()

        return carry

    lax.fori_loop(0, (NCHT + 7) // 8, body, 0)
    plsc.subcore_barrier()

    @pl.when(c == 0)
    def _():
        pltpu.sync_copy(deg_sh.at[pl.ds(s * RPT, RPT)],
                        out0_hbm.at[pl.ds(s * RPT, RPT)])

    @pl.when(c == 1)
    def _():
        pltpu.sync_copy(deg_sh.at[pl.ds(s * RPT, RPT)],
                        out1_hbm.at[pl.ds(s * RPT, RPT)])


_deg_call = pl.kernel(
    _deg_body,
    out_type=(jax.ShapeDtypeStruct((NPAD,), f32),
              jax.ShapeDtypeStruct((NPAD,), f32)),
    mesh=_mesh,
    scratch_types=[
        pltpu.VMEM_SHARED((NPAD,), f32),
        pltpu.VMEM((NCHT, CH), i32),
        pltpu.VMEM((CH,), f32),
        pltpu.SemaphoreType.DMA,
    ],
)


def _agg_body(g_hbm, srcf_hbm, dstr_hbm, zeros_hbm,
              out0_hbm, out1_hbm, s_sh, isrc, idst, rows,
              sem0, sem1, sem2, sem3):
    c = lax.axis_index("c")
    s = lax.axis_index("s")
    t = c * NS + s
    ept = NCHT * CH
    pltpu.sync_copy(zeros_hbm, s_sh.at[pl.ds(s * RPT, RPT)])
    pltpu.sync_copy(srcf_hbm.at[pl.ds(t * ept, ept)], isrc)
    pltpu.sync_copy(dstr_hbm.at[t], idst)
    plsc.subcore_barrier()

    sems_g = (sem0, sem1)
    sems_s = (sem2, sem3)
    pltpu.async_copy(g_hbm.at[isrc.at[pl.ds(0, CH)]], rows.at[0], sem0)

    def body(jj, carry):
        for b in range(2):
            j = jj * 2 + b
            nxt = j + 1

            @pl.when(nxt < NCHT)
            def _():
                @pl.when(nxt >= 2)
                def _():
                    pltpu.make_async_copy(
                        rows.at[1 - b], s_sh.at[idst.at[nxt - 2]],
                        sems_s[1 - b]).wait()

                pltpu.async_copy(g_hbm.at[isrc.at[pl.ds(nxt * CH, CH)]],
                                 rows.at[1 - b], sems_g[1 - b])

            @pl.when(j < NCHT)
            def _():
                pltpu.make_async_copy(
                    g_hbm.at[isrc.at[pl.ds(j * CH, CH)]],
                    rows.at[b], sems_g[b]).wait()
                pltpu.async_copy(rows.at[b], s_sh.at[idst.at[j]],
                                 sems_s[b], add=True)

        return carry

    lax.fori_loop(0, (NCHT + 1) // 2, body, 0)
    pltpu.make_async_copy(rows.at[(NCHT - 2) % 2],
                          s_sh.at[idst.at[NCHT - 2]],
                          sems_s[(NCHT - 2) % 2]).wait()
    pltpu.make_async_copy(rows.at[(NCHT - 1) % 2],
                          s_sh.at[idst.at[NCHT - 1]],
                          sems_s[(NCHT - 1) % 2]).wait()
    plsc.subcore_barrier()

    @pl.when(c == 0)
    def _():
        pltpu.sync_copy(s_sh.at[pl.ds(s * RPT, RPT)],
                        out0_hbm.at[pl.ds(s * RPT, RPT)])

    @pl.when(c == 1)
    def _():
        pltpu.sync_copy(s_sh.at[pl.ds(s * RPT, RPT)],
                        out1_hbm.at[pl.ds(s * RPT, RPT)])


_agg_call = pl.kernel(
    _agg_body,
    out_type=(jax.ShapeDtypeStruct((NPAD, DW), f32),
              jax.ShapeDtypeStruct((NPAD, DW), f32)),
    mesh=_mesh,
    scratch_types=[
        pltpu.VMEM_SHARED((NPAD, DW), f32),
        pltpu.VMEM((NCHT * CH,), i32),
        pltpu.VMEM((NCHT, CH), i32),
        pltpu.VMEM((2, CH, DW), f32),
        pltpu.SemaphoreType.DMA,
        pltpu.SemaphoreType.DMA,
        pltpu.SemaphoreType.DMA,
        pltpu.SemaphoreType.DMA,
    ],
)


def _k1_body(x_ref, d0_ref, d1_ref, w_ref, g_ref, dinv_ref):
    deg = d0_ref[...] + d1_ref[...] + 1.0
    dinv = lax.rsqrt(deg)
    h = jnp.dot(x_ref[...], w_ref[...], preferred_element_type=f32)
    g_ref[...] = h * dinv
    dinv_ref[...] = dinv


def _k1_call(x, d0, d1, w):
    return pl.pallas_call(
        _k1_body,
        grid=(G,),
        in_specs=[
            pl.BlockSpec((R, D_IN), lambda i: (i, 0)),
            pl.BlockSpec((R, 1), lambda i: (i, 0)),
            pl.BlockSpec((R, 1), lambda i: (i, 0)),
            pl.BlockSpec((D_IN, 2 * HID), lambda i: (0, 0)),
        ],
        out_specs=[
            pl.BlockSpec((R, DW), lambda i: (i, 0)),
            pl.BlockSpec((R, 1), lambda i: (i, 0)),
        ],
        out_shape=[
            jax.ShapeDtypeStruct((N, DW), f32),
            jax.ShapeDtypeStruct((N, 1), f32),
        ],
    )(x, d0, d1, w)


def _mid_body(dout, s0, s1, g, dinv, w, b, o_ref):
    dv = dinv[...]
    a = jnp.maximum(dv * (s0[...] + s1[...] + g[...]) + b[...], 0.0)
    h = jnp.dot(a, w[...], preferred_element_type=f32)
    g_next = h * dv
    if dout < DW:
        g_next = jnp.concatenate(
            [g_next, jnp.zeros((R, DW - dout), f32)], axis=1)
    o_ref[...] = g_next


def _mid_call(s0, s1, g, dinv, w, b, dout):
    return pl.pallas_call(
        functools.partial(_mid_body, dout),
        grid=(G,),
        in_specs=[
            pl.BlockSpec((R, DW), lambda i: (i, 0)),
            pl.BlockSpec((R, DW), lambda i: (i, 0)),
            pl.BlockSpec((R, DW), lambda i: (i, 0)),
            pl.BlockSpec((R, 1), lambda i: (i, 0)),
            pl.BlockSpec((DW, dout), lambda i: (0, 0)),
            pl.BlockSpec((1, DW), lambda i: (0, 0)),
        ],
        out_specs=pl.BlockSpec((R, DW), lambda i: (i, 0)),
        out_shape=jax.ShapeDtypeStruct((N, DW), f32),
    )(s0, s1, g, dinv, w, b.reshape(1, -1))


def _fin_body(s0, s1, g, dinv, b3, batch, w4, b4, out_ref, sums, cnts):
    i = pl.program_id(0)

    @pl.when(i == 0)
    def _():
        sums[...] = jnp.zeros((NUM_GRAPHS, HID), f32)
        cnts[...] = jnp.zeros((NUM_GRAPHS, 1), f32)

    dv = dinv[...]
    a = dv * (s0[...] + s1[...] + g[...])[:, :HID] + b3[...]
    oh = (batch[...] == lax.broadcasted_iota(i32, (R, NUM_GRAPHS), 1))
    oh = oh.astype(f32)
    sums[...] += lax.dot_general(oh, a, (((0,), (0,)), ((), ())),
                                 preferred_element_type=f32)
    cnts[...] += lax.dot_general(oh, jnp.ones((R, 1), f32),
                                 (((0,), (0,)), ((), ())),
                                 preferred_element_type=f32)

    @pl.when(i == G - 1)
    def _():
        pooled = sums[...] / jnp.maximum(cnts[...], 1.0)
        out_ref[...] = jnp.dot(pooled, w4[...],
                               preferred_element_type=f32) + b4[...]


def _fin_call(s0, s1, g, dinv, b3, batch, w4, b4):
    return pl.pallas_call(
        _fin_body,
        grid=(G,),
        in_specs=[
            pl.BlockSpec((R, DW), lambda i: (i, 0)),
            pl.BlockSpec((R, DW), lambda i: (i, 0)),
            pl.BlockSpec((R, DW), lambda i: (i, 0)),
            pl.BlockSpec((R, 1), lambda i: (i, 0)),
            pl.BlockSpec((1, HID), lambda i: (0, 0)),
            pl.BlockSpec((R, 1), lambda i: (i, 0)),
            pl.BlockSpec((HID, NUM_CLASSES), lambda i: (0, 0)),
            pl.BlockSpec((1, NUM_CLASSES), lambda i: (0, 0)),
        ],
        out_specs=pl.BlockSpec((NUM_GRAPHS, NUM_CLASSES), lambda i: (0, 0)),
        out_shape=jax.ShapeDtypeStruct((NUM_GRAPHS, NUM_CLASSES), f32),
        scratch_shapes=[
            pltpu.VMEM((NUM_GRAPHS, HID), f32),
            pltpu.VMEM((NUM_GRAPHS, 1), f32),
        ],
    )(s0, s1, g, dinv, b3.reshape(1, -1), batch, w4, b4.reshape(1, -1))


def kernel(x, edge_index, batch, W1, b1, W2, b2, W3, b3, W4, b4):
    src = edge_index[0].astype(i32)
    dst = edge_index[1].astype(i32).reshape(NC * NS, NCHT, CH)
    batch2 = batch.astype(i32).reshape(N, 1)
    z1 = jnp.zeros((RPT,), f32)
    zw = jnp.zeros((RPT, DW), f32)

    deg0, deg1 = _deg_call(dst, z1)
    g1, dinv = _k1_call(x, deg0.reshape(NPAD, 1),
                        deg1.reshape(NPAD, 1), W1)

    s1a, s1b = _agg_call(g1, src, dst, zw)
    g2 = _mid_call(s1a, s1b, g1, dinv, W2, b1, 2 * HID)

    s2a, s2b = _agg_call(g2, src, dst, zw)
    g3 = _mid_call(s2a, s2b, g2, dinv, W3, b2, HID)

    s3a, s3b = _agg_call(g3, src, dst, zw)
    out = _fin_call(s3a, s3b, g3, dinv, b3, batch2, W4, b4)
    return out


# speedup vs baseline: 27.6355x; 1.0115x over previous
import functools

import jax
import jax.numpy as jnp
from jax import lax
from jax.experimental import pallas as pl
from jax.experimental.pallas import tpu as pltpu
from jax.experimental.pallas import tpu_sc as plsc

N = 10000
E = 320000
D_IN = 128
HID = 64
NUM_CLASSES = 10
NUM_GRAPHS = 64

NC, NS = 2, 16
NPAD = 10240
RPT = NPAD // NS
CH = 80
NCHT = E // (NC * NS * CH)
DW = 128

R = 1000
G = N // R

_mesh = plsc.VectorSubcoreMesh(
    core_axis_name="c", subcore_axis_name="s", num_cores=NC, num_subcores=NS)

f32 = jnp.float32
i32 = jnp.int32


def _deg_body(dstr_hbm, zeros_hbm, out0_hbm, out1_hbm, deg_sh, idst, ones_v,
              sem):
    c = lax.axis_index("c")
    s = lax.axis_index("s")
    t = c * NS + s
    pltpu.sync_copy(zeros_hbm, deg_sh.at[pl.ds(s * RPT, RPT)])
    pltpu.sync_copy(dstr_hbm.at[t], idst)
    for k in range(CH // 16):
        ones_v[pl.ds(k * 16, 16)] = jnp.full((16,), 1.0, f32)
    plsc.subcore_barrier()

    def body(jj, carry):
        for b in range(8):
            j = jj * 8 + b

            @pl.when(j < NCHT)
            def _():
                pltpu.async_copy(ones_v, deg_sh.at[idst.at[j]], sem, add=True)

        for b in range(8):
            j = jj * 8 + b

            @pl.when(j < NCHT)
            def _():
                pltpu.make_async_copy(
                    ones_v, deg_sh.at[idst.at[j]], sem).wait()

        return carry

    lax.fori_loop(0, (NCHT + 7) // 8, body, 0)
    plsc.subcore_barrier()

    @pl.when(c == 0)
    def _():
        pltpu.sync_copy(deg_sh.at[pl.ds(s * RPT, RPT)],
                        out0_hbm.at[pl.ds(s * RPT, RPT)])

    @pl.when(c == 1)
    def _():
        pltpu.sync_copy(deg_sh.at[pl.ds(s * RPT, RPT)],
                        out1_hbm.at[pl.ds(s * RPT, RPT)])


_deg_call = pl.kernel(
    _deg_body,
    out_type=(jax.ShapeDtypeStruct((NPAD,), f32),
              jax.ShapeDtypeStruct((NPAD,), f32)),
    mesh=_mesh,
    scratch_types=[
        pltpu.VMEM_SHARED((NPAD,), f32),
        pltpu.VMEM((NCHT, CH), i32),
        pltpu.VMEM((CH,), f32),
        pltpu.SemaphoreType.DMA,
    ],
)


def _agg_body(g_hbm, srcf_hbm, dstr_hbm, zeros_hbm,
              out0_hbm, out1_hbm, s_sh, isrc, idst, rows,
              sem0, sem1, sem2, sem3):
    c = lax.axis_index("c")
    s = lax.axis_index("s")
    t = c * NS + s
    ept = NCHT * CH

    @pl.when(c == 0)
    def _():
        @pl.when(s < NS - 1)
        def _():
            pltpu.sync_copy(g_hbm.at[pl.ds(s * RPT, RPT)],
                            s_sh.at[pl.ds(s * RPT, RPT)])

        @pl.when(s == NS - 1)
        def _():
            pltpu.sync_copy(g_hbm.at[pl.ds(s * RPT, N - (NS - 1) * RPT)],
                            s_sh.at[pl.ds(s * RPT, N - (NS - 1) * RPT)])
            pltpu.sync_copy(zeros_hbm.at[pl.ds(0, NPAD - N)],
                            s_sh.at[pl.ds(N, NPAD - N)])

    @pl.when(c == 1)
    def _():
        pltpu.sync_copy(zeros_hbm, s_sh.at[pl.ds(s * RPT, RPT)])

    pltpu.sync_copy(srcf_hbm.at[pl.ds(t * ept, ept)], isrc)
    pltpu.sync_copy(dstr_hbm.at[t], idst)
    plsc.subcore_barrier()

    sems_g = (sem0, sem1)
    sems_s = (sem2, sem3)
    pltpu.async_copy(g_hbm.at[isrc.at[pl.ds(0, CH)]], rows.at[0], sem0)

    def body(jj, carry):
        for b in range(2):
            j = jj * 2 + b
            nxt = j + 1

            @pl.when(nxt < NCHT)
            def _():
                @pl.when(nxt >= 2)
                def _():
                    pltpu.make_async_copy(
                        rows.at[1 - b], s_sh.at[idst.at[nxt - 2]],
                        sems_s[1 - b]).wait()

                pltpu.async_copy(g_hbm.at[isrc.at[pl.ds(nxt * CH, CH)]],
                                 rows.at[1 - b], sems_g[1 - b])

            @pl.when(j < NCHT)
            def _():
                pltpu.make_async_copy(
                    g_hbm.at[isrc.at[pl.ds(j * CH, CH)]],
                    rows.at[b], sems_g[b]).wait()
                pltpu.async_copy(rows.at[b], s_sh.at[idst.at[j]],
                                 sems_s[b], add=True)

        return carry

    lax.fori_loop(0, (NCHT + 1) // 2, body, 0)
    pltpu.make_async_copy(rows.at[(NCHT - 2) % 2],
                          s_sh.at[idst.at[NCHT - 2]],
                          sems_s[(NCHT - 2) % 2]).wait()
    pltpu.make_async_copy(rows.at[(NCHT - 1) % 2],
                          s_sh.at[idst.at[NCHT - 1]],
                          sems_s[(NCHT - 1) % 2]).wait()
    plsc.subcore_barrier()

    @pl.when(c == 0)
    def _():
        pltpu.sync_copy(s_sh.at[pl.ds(s * RPT, RPT)],
                        out0_hbm.at[pl.ds(s * RPT, RPT)])

    @pl.when(c == 1)
    def _():
        pltpu.sync_copy(s_sh.at[pl.ds(s * RPT, RPT)],
                        out1_hbm.at[pl.ds(s * RPT, RPT)])


_agg_call = pl.kernel(
    _agg_body,
    out_type=(jax.ShapeDtypeStruct((NPAD, DW), f32),
              jax.ShapeDtypeStruct((NPAD, DW), f32)),
    mesh=_mesh,
    scratch_types=[
        pltpu.VMEM_SHARED((NPAD, DW), f32),
        pltpu.VMEM((NCHT * CH,), i32),
        pltpu.VMEM((NCHT, CH), i32),
        pltpu.VMEM((2, CH, DW), f32),
        pltpu.SemaphoreType.DMA,
        pltpu.SemaphoreType.DMA,
        pltpu.SemaphoreType.DMA,
        pltpu.SemaphoreType.DMA,
    ],
)


def _k1_body(x_ref, d0_ref, d1_ref, w_ref, g_ref, dinv_ref):
    deg = d0_ref[...] + d1_ref[...] + 1.0
    dinv = lax.rsqrt(deg)
    h = jnp.dot(x_ref[...], w_ref[...], preferred_element_type=f32)
    g_ref[...] = h * dinv
    dinv_ref[...] = dinv


def _k1_call(x, d0, d1, w):
    return pl.pallas_call(
        _k1_body,
        grid=(G,),
        in_specs=[
            pl.BlockSpec((R, D_IN), lambda i: (i, 0)),
            pl.BlockSpec((R, 1), lambda i: (i, 0)),
            pl.BlockSpec((R, 1), lambda i: (i, 0)),
            pl.BlockSpec((D_IN, 2 * HID), lambda i: (0, 0)),
        ],
        out_specs=[
            pl.BlockSpec((R, DW), lambda i: (i, 0)),
            pl.BlockSpec((R, 1), lambda i: (i, 0)),
        ],
        out_shape=[
            jax.ShapeDtypeStruct((N, DW), f32),
            jax.ShapeDtypeStruct((N, 1), f32),
        ],
    )(x, d0, d1, w)


def _mid_body(dout, s0, s1, dinv, w, b, o_ref):
    dv = dinv[...]
    a = jnp.maximum(dv * (s0[...] + s1[...]) + b[...], 0.0)
    h = jnp.dot(a, w[...], preferred_element_type=f32)
    g_next = h * dv
    if dout < DW:
        g_next = jnp.concatenate(
            [g_next, jnp.zeros((R, DW - dout), f32)], axis=1)
    o_ref[...] = g_next


def _mid_call(s0, s1, dinv, w, b, dout):
    return pl.pallas_call(
        functools.partial(_mid_body, dout),
        grid=(G,),
        in_specs=[
            pl.BlockSpec((R, DW), lambda i: (i, 0)),
            pl.BlockSpec((R, DW), lambda i: (i, 0)),
            pl.BlockSpec((R, 1), lambda i: (i, 0)),
            pl.BlockSpec((DW, dout), lambda i: (0, 0)),
            pl.BlockSpec((1, DW), lambda i: (0, 0)),
        ],
        out_specs=pl.BlockSpec((R, DW), lambda i: (i, 0)),
        out_shape=jax.ShapeDtypeStruct((N, DW), f32),
    )(s0, s1, dinv, w, b.reshape(1, -1))


def _fin_body(s0, s1, dinv, b3, batch, w4, b4, out_ref, sums, cnts):
    i = pl.program_id(0)

    @pl.when(i == 0)
    def _():
        sums[...] = jnp.zeros((NUM_GRAPHS, HID), f32)
        cnts[...] = jnp.zeros((NUM_GRAPHS, 1), f32)

    dv = dinv[...]
    a = dv * (s0[...] + s1[...])[:, :HID] + b3[...]
    oh = (batch[...] == lax.broadcasted_iota(i32, (R, NUM_GRAPHS), 1))
    oh = oh.astype(f32)
    sums[...] += lax.dot_general(oh, a, (((0,), (0,)), ((), ())),
                                 preferred_element_type=f32)
    cnts[...] += lax.dot_general(oh, jnp.ones((R, 1), f32),
                                 (((0,), (0,)), ((), ())),
                                 preferred_element_type=f32)

    @pl.when(i == G - 1)
    def _():
        pooled = sums[...] / jnp.maximum(cnts[...], 1.0)
        out_ref[...] = jnp.dot(pooled, w4[...],
                               preferred_element_type=f32) + b4[...]


def _fin_call(s0, s1, dinv, b3, batch, w4, b4):
    return pl.pallas_call(
        _fin_body,
        grid=(G,),
        in_specs=[
            pl.BlockSpec((R, DW), lambda i: (i, 0)),
            pl.BlockSpec((R, DW), lambda i: (i, 0)),
            pl.BlockSpec((R, 1), lambda i: (i, 0)),
            pl.BlockSpec((1, HID), lambda i: (0, 0)),
            pl.BlockSpec((R, 1), lambda i: (i, 0)),
            pl.BlockSpec((HID, NUM_CLASSES), lambda i: (0, 0)),
            pl.BlockSpec((1, NUM_CLASSES), lambda i: (0, 0)),
        ],
        out_specs=pl.BlockSpec((NUM_GRAPHS, NUM_CLASSES), lambda i: (0, 0)),
        out_shape=jax.ShapeDtypeStruct((NUM_GRAPHS, NUM_CLASSES), f32),
        scratch_shapes=[
            pltpu.VMEM((NUM_GRAPHS, HID), f32),
            pltpu.VMEM((NUM_GRAPHS, 1), f32),
        ],
    )(s0, s1, dinv, b3.reshape(1, -1), batch, w4, b4.reshape(1, -1))


def kernel(x, edge_index, batch, W1, b1, W2, b2, W3, b3, W4, b4):
    src = edge_index[0].astype(i32)
    dst = edge_index[1].astype(i32).reshape(NC * NS, NCHT, CH)
    batch2 = batch.astype(i32).reshape(N, 1)
    z1 = jnp.zeros((RPT,), f32)
    zw = jnp.zeros((RPT, DW), f32)

    deg0, deg1 = _deg_call(dst, z1)
    g1, dinv = _k1_call(x, deg0.reshape(NPAD, 1),
                        deg1.reshape(NPAD, 1), W1)

    s1a, s1b = _agg_call(g1, src, dst, zw)
    g2 = _mid_call(s1a, s1b, dinv, W2, b1, 2 * HID)

    s2a, s2b = _agg_call(g2, src, dst, zw)
    g3 = _mid_call(s2a, s2b, dinv, W3, b2, HID)

    s3a, s3b = _agg_call(g3, src, dst, zw)
    out = _fin_call(s3a, s3b, dinv, b3, batch2, W4, b4)
    return out
